# baseline (speedup 1.0000x reference)
import functools

import jax
import jax.numpy as jnp
from jax import lax
from jax.experimental import pallas as pl
from jax.experimental.pallas import tpu as pltpu
from jax.experimental.pallas import tpu_sc as plsc

B = 16384
F = 64
_INFO = plsc.get_sparse_core_info()
NC, NS, L = _INFO.num_cores, _INFO.num_subcores, _INFO.num_lanes
NW = NC * NS
BPW = B // NW
NCHUNK = BPW // 128
GROUPS = BPW // L


def _body(cell_idx_hbm, drug_idx_hbm, cell_fac_hbm, drug_fac_hbm,
          cell_b_hbm, drug_b_hbm, gb_hbm, out_hbm,
          cidx_v, didx_v, crow_v, drow_v, cbf_v, dbf_v, gb_v, pbuf_v, out_v, sem):
    wid = lax.axis_index("s") * NC + lax.axis_index("c")
    base = wid * BPW

    for j in range(NCHUNK):
        pltpu.sync_copy(cell_idx_hbm.at[pl.ds(base + j * 128, 128)], cidx_v.at[j])
        pltpu.sync_copy(drug_idx_hbm.at[pl.ds(base + j * 128, 128)], didx_v.at[j])
    pltpu.sync_copy(gb_hbm, gb_v)

    copies = []
    for j in range(NCHUNK):
        copies.append(pltpu.async_copy(
            cell_fac_hbm.at[cidx_v.at[j]], crow_v.at[pl.ds(j * 128, 128)], sem))
        copies.append(pltpu.async_copy(
            drug_fac_hbm.at[didx_v.at[j]], drow_v.at[pl.ds(j * 128, 128)], sem))
        copies.append(pltpu.async_copy(
            cell_b_hbm.at[cidx_v.at[j]], cbf_v.at[pl.ds(j * 128, 128)], sem))
        copies.append(pltpu.async_copy(
            drug_b_hbm.at[didx_v.at[j]], dbf_v.at[pl.ds(j * 128, 128)], sem))
    for c in copies:
        c.wait()

    iota = lax.broadcasted_iota(jnp.int32, (L,), 0)
    iota16 = iota * L
    gb = gb_v[...]

    def group(g, _):
        for rr in range(L):
            r = g * L + rr
            s = jnp.zeros((L,), jnp.float32)
            for k in range(F // L):
                c = crow_v[r, pl.ds(k * L, L)]
                d = drow_v[r, pl.ds(k * L, L)]
                s = s + c * d
            pbuf_v[pl.ds(rr * L, L)] = s
        acc = jnp.zeros((L,), jnp.float32)
        for j in range(L):
            rot = jnp.bitwise_and(iota + j, L - 1)
            acc = acc + plsc.load_gather(pbuf_v, [iota16 + rot])
        cb = cbf_v[pl.ds(g * L, L)]
        db = dbf_v[pl.ds(g * L, L)]
        out_v[pl.ds(g * L, L)] = acc + cb + db + gb
        return _

    lax.fori_loop(0, GROUPS, group, None)
    pltpu.sync_copy(out_v, out_hbm.at[pl.ds(base, BPW)])


@functools.partial(jax.jit, static_argnums=())
def kernel(cell_indices, drug_indices, cell_factors, drug_factors,
           cell_bias, drug_bias, global_bias):
    mesh = plsc.VectorSubcoreMesh(core_axis_name="c", subcore_axis_name="s")
    run = pl.kernel(
        _body, mesh=mesh,
        out_type=jax.ShapeDtypeStruct((B,), jnp.float32),
        scratch_types=[
            pltpu.VMEM((NCHUNK, 128), jnp.int32),
            pltpu.VMEM((NCHUNK, 128), jnp.int32),
            pltpu.VMEM((BPW, F), jnp.float32),
            pltpu.VMEM((BPW, F), jnp.float32),
            pltpu.VMEM((BPW,), jnp.float32),
            pltpu.VMEM((BPW,), jnp.float32),
            pltpu.VMEM((L,), jnp.float32),
            pltpu.VMEM((L * L,), jnp.float32),
            pltpu.VMEM((BPW,), jnp.float32),
            pltpu.SemaphoreType.DMA,
        ],
        compiler_params=pltpu.CompilerParams(
            needs_layout_passes=False, use_tc_tiling_on_sc=False),
    )
    return run(cell_indices.astype(jnp.int32), drug_indices.astype(jnp.int32),
               cell_factors, drug_factors,
               cell_bias.reshape(-1), drug_bias.reshape(-1),
               jnp.tile(global_bias, L))
